# Initial kernel scaffold; baseline (speedup 1.0000x reference)
import jax
import jax.numpy as jnp
from jax.experimental import pallas as pl


def kernel(hour, weekday, norm_time, hour_table, weekday_table):
    raise NotImplementedError("write your pallas kernel here")


# speedup vs baseline: 2.8307x; 2.8307x over previous
import functools

import jax
import jax.numpy as jnp
from jax import lax
from jax.experimental import pallas as pl
from jax.experimental.pallas import tpu as pltpu
from jax.experimental.pallas import tpu_sc as plsc

EMBED = 32
D_OUT = 70
B, L = 16384, 200
N = B * L
NC, NS = 2, 16
NW = NC * NS
N_W = N // NW
CH = 512
NCHUNK = N_W // CH
GRP = CH // 16

SIN_C = (6.283183465409584, -41.34148025958733, 81.59765524711814,
         -76.59489967393353, 41.26979637356445, -12.372272029175647)
COS_C = (0.9999999922855516, -19.739205552336067, 64.939172135788,
         -85.45116383102753, 60.176212682457354, -26.000455681228082,
         6.575502264032736)


def _horner(coeffs, z):
    r = jnp.float32(coeffs[-1])
    for c in coeffs[-2::-1]:
        r = r * z + jnp.float32(c)
    return r


def _sc_body(hour_hbm, wday_hbm, nt_hbm, ht_hbm, wt_hbm, out_hbm,
             ht_v, wt_v, h_v, w_v, t_v, out0_v, out1_v, sem0, sem1):
    wid = lax.axis_index("s") * NC + lax.axis_index("c")
    pltpu.sync_copy(ht_hbm, ht_v)
    pltpu.sync_copy(wt_hbm, wt_v)
    iota16 = lax.iota(jnp.int32, 16)
    row70 = iota16 * D_OUT

    def do_chunk(g, gg, out_v, sem_out):
        base = wid * N_W + g * CH
        pltpu.sync_copy(hour_hbm.at[pl.ds(base, CH)], h_v)
        pltpu.sync_copy(wday_hbm.at[pl.ds(base, CH)], w_v)
        pltpu.sync_copy(nt_hbm.at[pl.ds(base, CH)], t_v)

        def grp_body(j, _):
            h = h_v[pl.ds(j * 16, 16)]
            w = w_v[pl.ds(j * 16, 16)]
            t = t_v[pl.ds(j * 16, 16)]
            hidx = h * EMBED
            widx = w * EMBED
            obase = j * (16 * D_OUT) + row70
            for d in range(EMBED):
                val = plsc.load_gather(ht_v, [hidx + d])
                plsc.store_scatter(out_v, [obase + d], val)
            for d in range(EMBED):
                val = plsc.load_gather(wt_v, [widx + d])
                plsc.store_scatter(out_v, [obase + (EMBED + d)], val)
            x = t - lax.convert_element_type(
                lax.convert_element_type(t + 0.5, jnp.int32), jnp.float32)
            z = x * x
            s1 = x * _horner(SIN_C, z)
            c1 = _horner(COS_C, z)
            s2 = 2.0 * s1 * c1
            c2 = 1.0 - 2.0 * s1 * s1
            s4 = 2.0 * s2 * c2
            c4 = 1.0 - 2.0 * s2 * s2
            for k, val in enumerate((s1, c1, s2, c2, s4, c4)):
                plsc.store_scatter(out_v, [obase + (2 * EMBED + k)], val)
            return 0

        @pl.when(gg >= 1)
        def _():
            pltpu.make_async_copy(
                out_v,
                out_hbm.at[pl.ds((base - 2 * CH) * D_OUT, CH * D_OUT)],
                sem_out).wait()

        lax.fori_loop(0, GRP, grp_body, 0, unroll=2)
        pltpu.make_async_copy(
            out_v,
            out_hbm.at[pl.ds(base * D_OUT, CH * D_OUT)],
            sem_out).start()

    def chunk_pair(gg, _):
        do_chunk(gg * 2, gg, out0_v, sem0)
        do_chunk(gg * 2 + 1, gg, out1_v, sem1)
        return 0

    lax.fori_loop(0, NCHUNK // 2, chunk_pair, 0)
    for buf, (out_v, sem_out) in enumerate(((out0_v, sem0), (out1_v, sem1))):
        g = NCHUNK - 2 + buf
        base = wid * N_W + g * CH
        pltpu.make_async_copy(
            out_v,
            out_hbm.at[pl.ds(base * D_OUT, CH * D_OUT)],
            sem_out).wait()


@functools.partial(jax.jit, static_argnums=())
def _encode(hour_f, wday_f, nt_f, ht_flat, wt_flat):
    mesh = plsc.VectorSubcoreMesh(core_axis_name="c", subcore_axis_name="s")
    fn = pl.kernel(
        _sc_body,
        mesh=mesh,
        compiler_params=pltpu.CompilerParams(needs_layout_passes=False),
        out_type=jax.ShapeDtypeStruct((N * D_OUT,), jnp.float32),
        scratch_types=[
            pltpu.VMEM((24 * EMBED,), jnp.float32),
            pltpu.VMEM((7 * EMBED,), jnp.float32),
            pltpu.VMEM((CH,), jnp.int32),
            pltpu.VMEM((CH,), jnp.int32),
            pltpu.VMEM((CH,), jnp.float32),
            pltpu.VMEM((CH * D_OUT,), jnp.float32),
            pltpu.VMEM((CH * D_OUT,), jnp.float32),
            pltpu.SemaphoreType.DMA,
            pltpu.SemaphoreType.DMA,
        ],
    )
    return fn(hour_f, wday_f, nt_f, ht_flat, wt_flat)


def kernel(hour, weekday, norm_time, hour_table, weekday_table):
    hour_f = hour.reshape(N).astype(jnp.int32)
    wday_f = weekday.reshape(N).astype(jnp.int32)
    nt_f = norm_time.reshape(N)
    ht_flat = hour_table.reshape(24 * EMBED)
    wt_flat = weekday_table.reshape(7 * EMBED)
    out_flat = _encode(hour_f, wday_f, nt_f, ht_flat, wt_flat)
    return out_flat.reshape(B, L, D_OUT)


# speedup vs baseline: 5.1201x; 1.8088x over previous
import functools

import jax
import jax.numpy as jnp
from jax import lax
from jax.experimental import pallas as pl
from jax.experimental.pallas import tpu as pltpu
from jax.experimental.pallas import tpu_sc as plsc

EMBED = 32
D_OUT = 70
B, L = 16384, 200
N = B * L
NC, NS = 2, 16
NW = NC * NS
N_W = N // NW
CH = 256
NCHUNK = N_W // CH
GRP = CH // 16

SIN_C = (6.283183465409584, -41.34148025958733, 81.59765524711814,
         -76.59489967393353, 41.26979637356445, -12.372272029175647)
COS_C = (0.9999999922855516, -19.739205552336067, 64.939172135788,
         -85.45116383102753, 60.176212682457354, -26.000455681228082,
         6.575502264032736)


def _horner(coeffs, z):
    r = jnp.float32(coeffs[-1])
    for c in coeffs[-2::-1]:
        r = r * z + jnp.float32(c)
    return r


def _sc_body(hour_hbm, wday_hbm, nt_hbm, comb_hbm, out_hbm,
             comb_v, h_v, w_v, t_v, out0_v, out1_v, sem0, sem1):
    wid = lax.axis_index("s") * NC + lax.axis_index("c")
    pltpu.sync_copy(comb_hbm, comb_v)
    iota16 = lax.iota(jnp.int32, 16)

    def do_chunk(g, gg, out_v, sem_out):
        base = wid * N_W + g * CH
        pltpu.sync_copy(hour_hbm.at[pl.ds(base, CH)], h_v)
        pltpu.sync_copy(wday_hbm.at[pl.ds(base, CH)], w_v)
        pltpu.sync_copy(nt_hbm.at[pl.ds(base, CH)], t_v)

        def grp_body(j, _):
            h = h_v[pl.ds(j * 16, 16)]
            w = w_v[pl.ds(j * 16, 16)]
            t = t_v[pl.ds(j * 16, 16)]
            cidx = (h * 7 + w) * (2 * EMBED)
            orow = j * 16 + iota16

            @plsc.parallel_loop(0, 2 * EMBED, unroll=64)
            def _(d):
                val = plsc.load_gather(comb_v, [cidx + d])
                dcol = jnp.full((16,), d, jnp.int32)
                plsc.store_scatter(out_v, [orow, dcol], val)
            x = t - lax.convert_element_type(
                lax.convert_element_type(t + 0.5, jnp.int32), jnp.float32)
            z = x * x
            s1 = x * _horner(SIN_C, z)
            c1 = _horner(COS_C, z)
            s2 = 2.0 * s1 * c1
            c2 = 1.0 - 2.0 * s1 * s1
            s4 = 2.0 * s2 * c2
            c4 = 1.0 - 2.0 * s2 * s2
            for k, val in enumerate((s1, c1, s2, c2, s4, c4)):
                kcol = jnp.full((16,), 2 * EMBED + k, jnp.int32)
                plsc.store_scatter(out_v, [orow, kcol], val)
            return 0

        @pl.when(gg >= 1)
        def _():
            pltpu.make_async_copy(
                out_v,
                out_hbm.at[pl.ds(base - 2 * CH, CH), :],
                sem_out).wait()

        @plsc.parallel_loop(0, GRP)
        def _(j):
            grp_body(j, 0)
        pltpu.make_async_copy(
            out_v,
            out_hbm.at[pl.ds(base, CH), :],
            sem_out).start()

    def chunk_pair(gg, _):
        do_chunk(gg * 2, gg, out0_v, sem0)
        do_chunk(gg * 2 + 1, gg, out1_v, sem1)
        return 0

    lax.fori_loop(0, NCHUNK // 2, chunk_pair, 0)
    for buf, (out_v, sem_out) in enumerate(((out0_v, sem0), (out1_v, sem1))):
        g = NCHUNK - 2 + buf
        base = wid * N_W + g * CH
        pltpu.make_async_copy(
            out_v,
            out_hbm.at[pl.ds(base, CH), :],
            sem_out).wait()


@functools.partial(jax.jit, static_argnums=())
def _encode(hour_f, wday_f, nt_f, comb_flat):
    mesh = plsc.VectorSubcoreMesh(core_axis_name="c", subcore_axis_name="s")
    fn = pl.kernel(
        _sc_body,
        mesh=mesh,
        compiler_params=pltpu.CompilerParams(
            needs_layout_passes=False, use_tc_tiling_on_sc=True),
        out_type=jax.ShapeDtypeStruct((N, D_OUT), jnp.float32),
        scratch_types=[
            pltpu.VMEM((24 * 7 * 2 * EMBED,), jnp.float32),
            pltpu.VMEM((CH,), jnp.int32),
            pltpu.VMEM((CH,), jnp.int32),
            pltpu.VMEM((CH,), jnp.float32),
            pltpu.VMEM((CH, D_OUT), jnp.float32),
            pltpu.VMEM((CH, D_OUT), jnp.float32),
            pltpu.SemaphoreType.DMA,
            pltpu.SemaphoreType.DMA,
        ],
    )
    return fn(hour_f, wday_f, nt_f, comb_flat)


def kernel(hour, weekday, norm_time, hour_table, weekday_table):
    hour_f = hour.reshape(N).astype(jnp.int32)
    wday_f = weekday.reshape(N).astype(jnp.int32)
    nt_f = norm_time.reshape(N)
    comb_flat = jnp.concatenate([
        jnp.broadcast_to(hour_table[:, None, :], (24, 7, EMBED)),
        jnp.broadcast_to(weekday_table[None, :, :], (24, 7, EMBED)),
    ], axis=-1).reshape(24 * 7 * 2 * EMBED)
    out2d = _encode(hour_f, wday_f, nt_f, comb_flat)
    return out2d.reshape(B, L, D_OUT)
